# baseline (speedup 1.0000x reference)
import functools

import jax
import jax.numpy as jnp
from jax import lax
from jax.experimental import pallas as pl
from jax.experimental.pallas import tpu as pltpu
from jax.experimental.pallas import tpu_sc as plsc

N = 10000
D = 128
H = 128
O = 128
E = 320000

NTILE = 16
EPT = E // NTILE
CHUNK = 128
KC = 1
NBUF = 2
NCHUNK = 160
PAD = NCHUNK * CHUNK - EPT
RPT = N // NTILE
BN = 1000


def _mm_body(x0_ref, x1_ref, x2_ref, wg_ref, bg_ref, h0_ref, h1_ref, h2_ref):
    wg = wg_ref[...]
    bg = bg_ref[...]
    for x_ref, h_ref in ((x0_ref, h0_ref), (x1_ref, h1_ref), (x2_ref, h2_ref)):
        h = lax.dot_general(x_ref[...], wg, (((1,), (1,)), ((), ())),
                            preferred_element_type=jnp.float32) + bg
        h_ref[0, :, :] = h[:, :64]
        h_ref[1, :, :] = h[:, 64:]


def _matmul_h(x0, x1, x2, Wg, bg2):
    grid = (N // BN,)
    xspec = pl.BlockSpec((BN, D), lambda i: (i, 0))
    wspec = pl.BlockSpec((H, D), lambda i: (0, 0))
    bspec = pl.BlockSpec((1, H), lambda i: (0, 0))
    ospec = pl.BlockSpec((2, BN, 64), lambda i: (0, i, 0))
    out = jax.ShapeDtypeStruct((2, N, 64), jnp.float32)
    return pl.pallas_call(
        _mm_body,
        grid=grid,
        in_specs=[xspec, xspec, xspec, wspec, bspec],
        out_specs=[ospec, ospec, ospec],
        out_shape=[out, out, out],
    )(x0, x1, x2, Wg, bg2)


def _sc_body(h0, h1, h2, s0, s1, s2, d0, d1, d2, consts_hbm, onesr_hbm,
             acc0, acc1, acc2, degc,
             src_v, dst_v, rows_v, ones_v, acc_sh, deg_sh,
             sem0, sem1):
    sems = (sem0, sem1)
    c = lax.axis_index("c")
    s = lax.axis_index("s")
    row0 = s * RPT
    NOP = NCHUNK // KC
    half_ops = NOP // 2

    pltpu.sync_copy(onesr_hbm, ones_v)

    for t, (h_t, s_t, d_t, a_t) in enumerate(
            ((h0, s0, d0, acc0), (h1, s1, d1, acc1), (h2, s2, d2, acc2))):
        pltpu.sync_copy(h_t.at[pl.ds(c * N + row0, RPT)],
                        acc_sh.at[pl.ds(row0, RPT)])
        pltpu.sync_copy(consts_hbm.at[c], deg_sh.at[pl.ds(row0, RPT)])
        pltpu.sync_copy(s_t.at[c, s], src_v)
        pltpu.sync_copy(d_t.at[s], dst_v)
        plsc.subcore_barrier()

        def do_op(q, b, sem):
            pltpu.make_async_copy(h_t.at[src_v.at[q]], rows_v.at[b],
                                  sem).wait()
            pltpu.sync_copy(rows_v.at[b], acc_sh.at[dst_v.at[q]], add=True)

            @pl.when((q < half_ops) == (c == 0))
            def _():
                pltpu.sync_copy(ones_v, deg_sh.at[dst_v.at[q]], add=True)

            @pl.when(q < NOP - NBUF)
            def _():
                pltpu.async_copy(h_t.at[src_v.at[q + NBUF]], rows_v.at[b],
                                 sem)

        for b in range(NBUF):
            pltpu.async_copy(h_t.at[src_v.at[b]], rows_v.at[b], sems[b])

        def rnd(i, carry):
            for b in range(NBUF):
                do_op(NBUF * i + b, b, sems[b])
            return carry

        lax.fori_loop(0, NOP // NBUF, rnd, 0)
        plsc.subcore_barrier()

        pltpu.sync_copy(acc_sh.at[pl.ds(row0, RPT)],
                        a_t.at[c, pl.ds(row0, RPT)])
        pltpu.sync_copy(deg_sh.at[pl.ds(row0, RPT)],
                        degc.at[c, t, pl.ds(row0, RPT)])


def _sc_scatter(h0, h1, h2, s0, s1, s2, d0, d1, d2, consts_hbm, onesr_hbm):
    mesh = plsc.VectorSubcoreMesh(core_axis_name="c", subcore_axis_name="s")
    acc = jax.ShapeDtypeStruct((2, N, 64), jnp.float32)
    deg = jax.ShapeDtypeStruct((2, 3, N, 16), jnp.float32)
    fn = pl.kernel(
        _sc_body,
        out_type=(acc, acc, acc, deg),
        mesh=mesh,
        scratch_types=[
            pltpu.VMEM((NCHUNK, CHUNK), jnp.int32),
            pltpu.VMEM((NCHUNK, CHUNK), jnp.int32),
            pltpu.VMEM((NBUF, CHUNK, 64), jnp.float32),
            pltpu.VMEM((CHUNK, 16), jnp.float32),
            pltpu.VMEM_SHARED((N + 8, 64), jnp.float32),
            pltpu.VMEM_SHARED((N + 8, 16), jnp.float32),
            pltpu.SemaphoreType.DMA,
            pltpu.SemaphoreType.DMA,
        ],
        compiler_params=pltpu.CompilerParams(use_tc_tiling_on_sc=False),
    )
    return fn(h0, h1, h2, s0, s1, s2, d0, d1, d2, consts_hbm, onesr_hbm)


def _ng_body(a0_ref, a1_ref, a2_ref, deg_ref, wih_ref, whh_ref, bih_ref,
             bhh_ref, wc_ref, bc_ref, out_ref, zsum_ref):
    i = pl.program_id(0)
    zs = []
    for t, a_ref in enumerate((a0_ref, a1_ref, a2_ref)):
        a = jnp.concatenate([a_ref[0], a_ref[1]], axis=1)
        d = deg_ref[0, t, :, 0:1] + deg_ref[1, t, :, 0:1]
        z = jnp.maximum(a / d, 0.0)
        zs.append(jnp.sum(z, axis=0, keepdims=True))
    zsum = jnp.concatenate(zs, axis=0)

    @pl.when(i == 0)
    def _():
        zsum_ref[...] = zsum

    @pl.when(i != 0)
    def _():
        zsum_ref[...] = zsum_ref[...] + zsum

    @pl.when(i == N // BN - 1)
    def _():
        zm = zsum_ref[...] * (1.0 / N)
        wih = wih_ref[...]
        whh = whh_ref[...]
        bih = bih_ref[...]
        bhh = bhh_ref[...]
        h = jnp.zeros((1, H), jnp.float32)
        for t in range(3):
            x = zm[t:t + 1, :]
            gi = lax.dot_general(x, wih, (((1,), (1,)), ((), ())),
                                 preferred_element_type=jnp.float32) + bih
            gh = lax.dot_general(h, whh, (((1,), (1,)), ((), ())),
                                 preferred_element_type=jnp.float32) + bhh
            i_r, i_z, i_n = gi[:, :H], gi[:, H:2 * H], gi[:, 2 * H:]
            h_r, h_z, h_n = gh[:, :H], gh[:, H:2 * H], gh[:, 2 * H:]
            r = jax.nn.sigmoid(i_r + h_r)
            zz = jax.nn.sigmoid(i_z + h_z)
            n = jnp.tanh(i_n + r * h_n)
            h = (1.0 - zz) * n + zz * h
        out_ref[...] = lax.dot_general(
            h, wc_ref[...], (((1,), (1,)), ((), ())),
            preferred_element_type=jnp.float32) + bc_ref[...]


def _norm_gru(acc0, acc1, acc2, degc, Wih, Whh, bih2, bhh2, Wc, bc2):
    grid = (N // BN,)
    aspec = pl.BlockSpec((2, BN, 64), lambda i: (0, i, 0))
    dspec = pl.BlockSpec((2, 3, BN, 16), lambda i: (0, 0, i, 0))

    def full(shape):
        return pl.BlockSpec(shape, lambda i: tuple(0 for _ in shape))

    return pl.pallas_call(
        _ng_body,
        grid=grid,
        in_specs=[aspec, aspec, aspec, dspec,
                  full((3 * H, H)), full((3 * H, H)), full((1, 3 * H)),
                  full((1, 3 * H)), full((H, H)), full((1, O))],
        out_specs=pl.BlockSpec((1, O), lambda i: (0, 0)),
        out_shape=jax.ShapeDtypeStruct((1, O), jnp.float32),
        scratch_shapes=[pltpu.VMEM((3, H), jnp.float32)],
    )(acc0, acc1, acc2, degc, Wih, Whh, bih2, bhh2, Wc, bc2)


def kernel(x0, ei0, x1, ei1, x2, ei2, Wg, bg, Wih, Whh, bih, bhh, Wc, bc):
    h3 = _matmul_h(x0, x1, x2, Wg, bg.reshape(1, H))
    hs = [h.reshape(2 * N, 64) for h in h3]

    srcs, dsts = [], []
    for ei in (ei0, ei1, ei2):
        src = jnp.pad(ei[0].reshape(NTILE, EPT), ((0, 0), (0, PAD)))
        src = src.reshape(NTILE, NCHUNK, CHUNK)
        srcs.append(jnp.stack([src, src + N]))
        dst = jnp.pad(ei[1].reshape(NTILE, EPT), ((0, 0), (0, PAD)),
                      constant_values=N)
        dsts.append(dst.reshape(NTILE, NCHUNK, CHUNK))

    consts_hbm = jnp.stack([jnp.ones((RPT, 16), jnp.float32),
                            jnp.zeros((RPT, 16), jnp.float32)])
    onesr_hbm = jnp.ones((CHUNK, 16), jnp.float32)
    acc0, acc1, acc2, degc = _sc_scatter(hs[0], hs[1], hs[2],
                                         srcs[0], srcs[1], srcs[2],
                                         dsts[0], dsts[1], dsts[2],
                                         consts_hbm, onesr_hbm)
    return _norm_gru(acc0, acc1, acc2, degc, Wih, Whh,
                     bih.reshape(1, 3 * H), bhh.reshape(1, 3 * H),
                     Wc, bc.reshape(1, O))


# speedup vs baseline: 1.3227x; 1.3227x over previous
import functools

import jax
import jax.numpy as jnp
from jax import lax
from jax.experimental import pallas as pl
from jax.experimental.pallas import tpu as pltpu
from jax.experimental.pallas import tpu_sc as plsc

N = 10000
D = 128
H = 128
O = 128
E = 320000

NTILE = 16
EPT = E // NTILE
CHUNK = 128
KC = 1
NBUF = 2
NCHUNK = 158
PAD = NCHUNK * CHUNK - EPT
RPT = N // NTILE
BN = 1000


def _mm_body(x0_ref, x1_ref, x2_ref, wg_ref, bg_ref, h0_ref, h1_ref, h2_ref):
    wg = wg_ref[...]
    bg = bg_ref[...]
    for x_ref, h_ref in ((x0_ref, h0_ref), (x1_ref, h1_ref), (x2_ref, h2_ref)):
        h = lax.dot_general(x_ref[...], wg, (((1,), (1,)), ((), ())),
                            preferred_element_type=jnp.float32) + bg
        h_ref[0, :, :] = h[:, :64]
        h_ref[1, :, :] = h[:, 64:]


def _matmul_h(x0, x1, x2, Wg, bg2):
    grid = (N // BN,)
    xspec = pl.BlockSpec((BN, D), lambda i: (i, 0))
    wspec = pl.BlockSpec((H, D), lambda i: (0, 0))
    bspec = pl.BlockSpec((1, H), lambda i: (0, 0))
    ospec = pl.BlockSpec((2, BN, 64), lambda i: (0, i, 0))
    out = jax.ShapeDtypeStruct((2, N, 64), jnp.float32)
    return pl.pallas_call(
        _mm_body,
        grid=grid,
        in_specs=[xspec, xspec, xspec, wspec, bspec],
        out_specs=[ospec, ospec, ospec],
        out_shape=[out, out, out],
    )(x0, x1, x2, Wg, bg2)


def _sc_body(h0, h1, h2, s0, s1, s2, d0, d1, d2, consts_hbm, onesr_hbm,
             acc0, acc1, acc2, degc,
             src_v, dst_v, rows_v, ones_v, acc_sh, deg_sh,
             sem0, sem1):
    sems = (sem0, sem1)
    c = lax.axis_index("c")
    s = lax.axis_index("s")
    row0 = s * RPT
    NOP = NCHUNK // KC
    half_ops = NOP // 2

    pltpu.sync_copy(onesr_hbm, ones_v)

    for t, (h_t, s_t, d_t, a_t) in enumerate(
            ((h0, s0, d0, acc0), (h1, s1, d1, acc1), (h2, s2, d2, acc2))):
        pltpu.sync_copy(h_t.at[pl.ds(c * N + row0, RPT)],
                        acc_sh.at[pl.ds(row0, RPT)])
        pltpu.sync_copy(consts_hbm.at[c], deg_sh.at[pl.ds(row0, RPT)])
        pltpu.sync_copy(s_t.at[c, s], src_v)
        pltpu.sync_copy(d_t.at[s], dst_v)
        plsc.subcore_barrier()

        def do_op(q, b, sem):
            pltpu.make_async_copy(h_t.at[src_v.at[q]], rows_v.at[b],
                                  sem).wait()
            pltpu.sync_copy(rows_v.at[b], acc_sh.at[dst_v.at[q]], add=True)

            @pl.when((q < half_ops) == (c == 0))
            def _():
                pltpu.sync_copy(ones_v, deg_sh.at[dst_v.at[q]], add=True)

            @pl.when(q < NOP - NBUF)
            def _():
                pltpu.async_copy(h_t.at[src_v.at[q + NBUF]], rows_v.at[b],
                                 sem)

        for b in range(NBUF):
            pltpu.async_copy(h_t.at[src_v.at[b]], rows_v.at[b], sems[b])

        def rnd(i, carry):
            for b in range(NBUF):
                do_op(NBUF * i + b, b, sems[b])
            return carry

        lax.fori_loop(0, NOP // NBUF, rnd, 0)
        plsc.subcore_barrier()

        pltpu.sync_copy(acc_sh.at[pl.ds(row0, RPT)],
                        a_t.at[c, pl.ds(row0, RPT)])
        pltpu.sync_copy(deg_sh.at[pl.ds(row0, RPT)],
                        degc.at[c, t, pl.ds(row0, RPT)])


def _sc_scatter(h0, h1, h2, s0, s1, s2, d0, d1, d2, consts_hbm, onesr_hbm):
    mesh = plsc.VectorSubcoreMesh(core_axis_name="c", subcore_axis_name="s")
    acc = jax.ShapeDtypeStruct((2, N, 64), jnp.float32)
    deg = jax.ShapeDtypeStruct((2, 3, N, 16), jnp.float32)
    fn = pl.kernel(
        _sc_body,
        out_type=(acc, acc, acc, deg),
        mesh=mesh,
        scratch_types=[
            pltpu.VMEM((NCHUNK, CHUNK), jnp.int32),
            pltpu.VMEM((NCHUNK, CHUNK), jnp.int32),
            pltpu.VMEM((NBUF, CHUNK, 64), jnp.float32),
            pltpu.VMEM((CHUNK, 16), jnp.float32),
            pltpu.VMEM_SHARED((N + 8, 64), jnp.float32),
            pltpu.VMEM_SHARED((N + 8, 16), jnp.float32),
            pltpu.SemaphoreType.DMA,
            pltpu.SemaphoreType.DMA,
        ],
        compiler_params=pltpu.CompilerParams(use_tc_tiling_on_sc=False),
    )
    return fn(h0, h1, h2, s0, s1, s2, d0, d1, d2, consts_hbm, onesr_hbm)


def _norm_body(a0_ref, a1_ref, a2_ref, deg_ref, out_ref):
    i = pl.program_id(0)
    zs = []
    for t, a_ref in enumerate((a0_ref, a1_ref, a2_ref)):
        a = jnp.concatenate([a_ref[0], a_ref[1]], axis=1)
        d = deg_ref[0, t, :, 0:1] + deg_ref[1, t, :, 0:1]
        z = jnp.maximum(a / d, 0.0)
        zs.append(jnp.sum(z, axis=0, keepdims=True))
    zsum = jnp.concatenate(zs, axis=0)

    @pl.when(i == 0)
    def _():
        out_ref[...] = zsum

    @pl.when(i != 0)
    def _():
        out_ref[...] = out_ref[...] + zsum


def _normalize(acc0, acc1, acc2, degc):
    grid = (N // BN,)
    aspec = pl.BlockSpec((2, BN, 64), lambda i: (0, i, 0))
    dspec = pl.BlockSpec((2, 3, BN, 16), lambda i: (0, 0, i, 0))
    ospec = pl.BlockSpec((3, H), lambda i: (0, 0))
    return pl.pallas_call(
        _norm_body,
        grid=grid,
        in_specs=[aspec, aspec, aspec, dspec],
        out_specs=ospec,
        out_shape=jax.ShapeDtypeStruct((3, H), jnp.float32),
    )(acc0, acc1, acc2, degc)


def _gru_body(zsum_ref, wih_ref, whh_ref, bih_ref, bhh_ref, wc_ref, bc_ref,
              out_ref):
    zm = zsum_ref[...] * (1.0 / N)
    wih = wih_ref[...]
    whh = whh_ref[...]
    bih = bih_ref[...]
    bhh = bhh_ref[...]
    h = jnp.zeros((1, H), jnp.float32)
    for t in range(3):
        x = zm[t:t + 1, :]
        gi = lax.dot_general(x, wih, (((1,), (1,)), ((), ())),
                             preferred_element_type=jnp.float32) + bih
        gh = lax.dot_general(h, whh, (((1,), (1,)), ((), ())),
                             preferred_element_type=jnp.float32) + bhh
        i_r, i_z, i_n = gi[:, :H], gi[:, H:2 * H], gi[:, 2 * H:]
        h_r, h_z, h_n = gh[:, :H], gh[:, H:2 * H], gh[:, 2 * H:]
        r = jax.nn.sigmoid(i_r + h_r)
        z = jax.nn.sigmoid(i_z + h_z)
        n = jnp.tanh(i_n + r * h_n)
        h = (1.0 - z) * n + z * h
    out_ref[...] = lax.dot_general(h, wc_ref[...], (((1,), (1,)), ((), ())),
                                   preferred_element_type=jnp.float32) \
        + bc_ref[...]


def _gru(zsum, Wih, Whh, bih2, bhh2, Wc, bc2):
    return pl.pallas_call(
        _gru_body,
        out_shape=jax.ShapeDtypeStruct((1, O), jnp.float32),
    )(zsum, Wih, Whh, bih2, bhh2, Wc, bc2)


def kernel(x0, ei0, x1, ei1, x2, ei2, Wg, bg, Wih, Whh, bih, bhh, Wc, bc):
    h3 = _matmul_h(x0, x1, x2, Wg, bg.reshape(1, H))
    hs = [h.reshape(2 * N, 64) for h in h3]

    srcs, dsts = [], []
    for ei in (ei0, ei1, ei2):
        src = jnp.pad(ei[0].reshape(NTILE, EPT), ((0, 0), (0, PAD)))
        src = src.reshape(NTILE, NCHUNK, CHUNK)
        srcs.append(jnp.stack([src, src + N]))
        dst = jnp.pad(ei[1].reshape(NTILE, EPT), ((0, 0), (0, PAD)),
                      constant_values=N)
        dsts.append(dst.reshape(NTILE, NCHUNK, CHUNK))

    consts_hbm = jnp.stack([jnp.ones((RPT, 16), jnp.float32),
                            jnp.zeros((RPT, 16), jnp.float32)])
    onesr_hbm = jnp.ones((CHUNK, 16), jnp.float32)
    acc0, acc1, acc2, degc = _sc_scatter(hs[0], hs[1], hs[2],
                                         srcs[0], srcs[1], srcs[2],
                                         dsts[0], dsts[1], dsts[2],
                                         consts_hbm, onesr_hbm)
    zsum = _normalize(acc0, acc1, acc2, degc)
    return _gru(zsum, Wih, Whh, bih.reshape(1, 3 * H), bhh.reshape(1, 3 * H),
                Wc, bc.reshape(1, O))
